# baseline (speedup 1.0000x reference)
import functools

import jax
import jax.numpy as jnp
from jax import lax
from jax.experimental import pallas as pl
from jax.experimental.pallas import tpu as pltpu
from jax.experimental.pallas import tpu_sc as plsc

E = 8
D = 768
H = 512
T = 2048
CAP = 1024.0
BT = 128
PT = T + E * BT
NB = PT // BT
BE_ROWS = 32
KWW = 128
DW = D + KWW
NC = 2
NS = 16
NW = NC * NS
TW = T // NW


def _cumsum1(a):
    n = a.shape[1]
    d = 1
    while d < n:
        z = jnp.zeros((a.shape[0], d), a.dtype)
        a = a + jnp.concatenate([z, a[:, : n - d]], axis=1)
        d *= 2
    return a


def _route_kernel(x_ref, wr_ref, br_ref, dst_ref, kw_ref, be_ref):
    st = lax.dot_general(
        wr_ref[...], x_ref[...], (((0,), (1,)), ((), ())),
        preferred_element_type=jnp.float32)
    st = st + br_ref[...]
    iota = lax.broadcasted_iota(jnp.int32, (E, T), 0)
    v0 = jnp.max(st, axis=0, keepdims=True)
    e0 = jnp.min(jnp.where(st == v0, iota, E), axis=0, keepdims=True)
    masked = jnp.where(iota == e0, -jnp.inf, st)
    v1 = jnp.max(masked, axis=0, keepdims=True)
    e1 = jnp.min(jnp.where(masked == v1, iota, E), axis=0, keepdims=True)
    s0 = 1.0 / (1.0 + jnp.exp(v1 - v0))
    s1 = 1.0 - s0
    oh0 = (iota == e0).astype(jnp.float32)
    oh1 = (iota == e1).astype(jnp.float32)
    c0 = _cumsum1(oh0)
    c1 = _cumsum1(oh1)
    pos0 = jnp.sum(c0 * oh0, axis=0, keepdims=True)
    pos1 = jnp.sum((c0 + c1) * oh1, axis=0, keepdims=True)
    m0 = pos0 < CAP
    m1 = pos1 < CAP
    kw = jnp.where(m0, s0, 0.0) + jnp.where(m1, s1, 0.0)
    kw_ref[...] = jnp.broadcast_to(jnp.transpose(kw, (1, 0)), (T, KWW))
    g = jnp.maximum(jnp.where(m0, e0, 0), jnp.where(m1, e1, 0))
    ohg = (iota == g).astype(jnp.float32)
    cg = _cumsum1(ohg)
    rank = jnp.sum(cg * ohg, axis=0, keepdims=True) - 1.0
    cnt = cg[:, T - 1 : T]
    pc = jnp.ceil(cnt * (1.0 / BT)) * BT
    tri = (
        lax.broadcasted_iota(jnp.int32, (E, E), 0)
        > lax.broadcasted_iota(jnp.int32, (E, E), 1)
    ).astype(jnp.float32)
    starts = jnp.dot(tri, pc, preferred_element_type=jnp.float32)
    dstf = jnp.sum(ohg * starts, axis=0, keepdims=True) + rank
    dst_ref[...] = dstf.astype(jnp.int32)
    ends = starts + pc
    jb = (lax.broadcasted_iota(jnp.int32, (1, BE_ROWS), 1) * BT).astype(
        jnp.float32)
    be = jnp.sum((jb >= ends).astype(jnp.int32), axis=0, keepdims=True)
    be = jnp.minimum(be, E - 1)
    nact = (ends[E - 1, 0] * (1.0 / BT)).astype(jnp.int32)
    lanes = lax.broadcasted_iota(jnp.int32, (1, BE_ROWS), 1)
    be_ref[...] = jnp.where(lanes == NB, nact, be)


def _ffn_kernel(be_ref, xs_ref, w1_ref, w2_ref, w3_ref, ys_ref):
    @pl.when(pl.program_id(0) < be_ref[NB])
    def _():
        xb = xs_ref[:, 0:D]
        h1 = jnp.dot(xb, w1_ref[0], preferred_element_type=jnp.float32)
        h2 = jnp.dot(xb, w2_ref[0], preferred_element_type=jnp.float32)
        z = h2 * h1
        hh = 0.5 * z * (1.0 + lax.erf(z * 0.7071067811865476))
        y = jnp.dot(hh, w3_ref[0], preferred_element_type=jnp.float32)
        kwb = xs_ref[:, D : D + 1]
        ys_ref[...] = kwb * y + (1.0 - kwb) * xb


@functools.lru_cache(maxsize=1)
def _make_sc_kernels():
    mesh = plsc.VectorSubcoreMesh(core_axis_name="c", subcore_axis_name="s")

    @functools.partial(
        pl.kernel,
        mesh=mesh,
        out_type=jax.ShapeDtypeStruct((PT, DW), jnp.float32),
        scratch_types=[
            pltpu.VMEM((TW,), jnp.int32),
            pltpu.VMEM((TW, DW), jnp.float32),
            pltpu.SemaphoreType.DMA,
        ],
    )
    def sc_scatter(dst_hbm, x_hbm, kw16_hbm, xs_hbm, idx_v, rows_v, sem):
        wid = lax.axis_index("s") * NC + lax.axis_index("c")
        base = wid * TW
        pltpu.sync_copy(dst_hbm.at[0, pl.ds(base, TW)], idx_v)
        pltpu.sync_copy(x_hbm.at[pl.ds(base, TW)], rows_v.at[:, pl.ds(0, D)])
        pltpu.sync_copy(kw16_hbm.at[pl.ds(base, TW)],
                        rows_v.at[:, pl.ds(D, KWW)])
        pltpu.async_copy(rows_v, xs_hbm.at[idx_v], sem).wait()

    @functools.partial(
        pl.kernel,
        mesh=mesh,
        out_type=jax.ShapeDtypeStruct((T, D), jnp.float32),
        scratch_types=[
            pltpu.VMEM((TW,), jnp.int32),
            pltpu.VMEM((TW, D), jnp.float32),
            pltpu.SemaphoreType.DMA,
        ],
    )
    def sc_gather(dst_hbm, ys_hbm, g_hbm, idx_v, rows_v, sem):
        wid = lax.axis_index("s") * NC + lax.axis_index("c")
        base = wid * TW
        pltpu.sync_copy(dst_hbm.at[0, pl.ds(base, TW)], idx_v)
        pltpu.async_copy(ys_hbm.at[idx_v], rows_v, sem).wait()
        pltpu.sync_copy(rows_v, g_hbm.at[pl.ds(base, TW)])

    return sc_scatter, sc_gather


def kernel(x, Wr, br, w1, w2, w3):
    b, t, d = x.shape
    x_flat = x.reshape(T, D)

    dst2, kw16, be2 = pl.pallas_call(
        _route_kernel,
        out_shape=[
            jax.ShapeDtypeStruct((1, T), jnp.int32),
            jax.ShapeDtypeStruct((T, KWW), jnp.float32),
            jax.ShapeDtypeStruct((1, BE_ROWS), jnp.int32),
        ],
    )(x_flat, Wr, br.reshape(E, 1))

    sc_scatter, sc_gather = _make_sc_kernels()
    xs = sc_scatter(dst2, x_flat, kw16)

    grid_spec = pltpu.PrefetchScalarGridSpec(
        num_scalar_prefetch=1,
        grid=(NB,),
        in_specs=[
            pl.BlockSpec((BT, DW), lambda j, be: (j, 0)),
            pl.BlockSpec((1, D, H), lambda j, be: (be[j], 0, 0)),
            pl.BlockSpec((1, D, H), lambda j, be: (be[j], 0, 0)),
            pl.BlockSpec((1, H, D), lambda j, be: (be[j], 0, 0)),
        ],
        out_specs=pl.BlockSpec((BT, D), lambda j, be: (j, 0)),
    )
    ys = pl.pallas_call(
        _ffn_kernel,
        grid_spec=grid_spec,
        out_shape=jax.ShapeDtypeStruct((PT, D), jnp.float32),
    )(be2.reshape(BE_ROWS), xs, w1, w2, w3)

    out = sc_gather(dst2, ys)
    return out.reshape(b, t, d)


# speedup vs baseline: 1.0162x; 1.0162x over previous
import functools

import jax
import jax.numpy as jnp
from jax import lax
from jax.experimental import pallas as pl
from jax.experimental.pallas import tpu as pltpu
from jax.experimental.pallas import tpu_sc as plsc

E = 8
D = 768
H = 512
T = 2048
CAP = 1024.0
BT = 128
PT = T + E * BT
NB = PT // BT
BE_ROWS = 32
KWW = 128
DW = D + KWW
NC = 2
NS = 16
NW = NC * NS
TW = T // NW


def _cumsum1(a):
    n = a.shape[1]
    d = 1
    while d < n:
        z = jnp.zeros((a.shape[0], d), a.dtype)
        a = a + jnp.concatenate([z, a[:, : n - d]], axis=1)
        d *= 2
    return a


def _route_kernel(x_ref, wr_ref, br_ref, dst_ref, kw_ref, be_ref):
    st = lax.dot_general(
        wr_ref[...], x_ref[...], (((0,), (1,)), ((), ())),
        preferred_element_type=jnp.float32)
    st = st + br_ref[...]
    iota = lax.broadcasted_iota(jnp.int32, (E, T), 0)
    v0 = jnp.max(st, axis=0, keepdims=True)
    e0 = jnp.min(jnp.where(st == v0, iota, E), axis=0, keepdims=True)
    masked = jnp.where(iota == e0, -jnp.inf, st)
    v1 = jnp.max(masked, axis=0, keepdims=True)
    e1 = jnp.min(jnp.where(masked == v1, iota, E), axis=0, keepdims=True)
    s0 = 1.0 / (1.0 + jnp.exp(v1 - v0))
    s1 = 1.0 - s0
    oh0 = (iota == e0).astype(jnp.float32)
    oh1 = (iota == e1).astype(jnp.float32)
    c0 = _cumsum1(oh0)
    c1 = _cumsum1(oh1)
    pos0 = jnp.sum(c0 * oh0, axis=0, keepdims=True)
    pos1 = jnp.sum((c0 + c1) * oh1, axis=0, keepdims=True)
    m0 = pos0 < CAP
    m1 = pos1 < CAP
    kw = jnp.where(m0, s0, 0.0) + jnp.where(m1, s1, 0.0)
    kw_ref[...] = jnp.broadcast_to(jnp.transpose(kw, (1, 0)), (T, KWW))
    g = jnp.maximum(jnp.where(m0, e0, 0), jnp.where(m1, e1, 0))
    ohg = (iota == g).astype(jnp.float32)
    cg = _cumsum1(ohg)
    rank = jnp.sum(cg * ohg, axis=0, keepdims=True) - 1.0
    cnt = cg[:, T - 1 : T]
    pc = jnp.ceil(cnt * (1.0 / BT)) * BT
    tri = (
        lax.broadcasted_iota(jnp.int32, (E, E), 0)
        > lax.broadcasted_iota(jnp.int32, (E, E), 1)
    ).astype(jnp.float32)
    starts = jnp.dot(tri, pc, preferred_element_type=jnp.float32)
    dstf = jnp.sum(ohg * starts, axis=0, keepdims=True) + rank
    dst_ref[...] = dstf.astype(jnp.int32)
    ends = starts + pc
    jb = (lax.broadcasted_iota(jnp.int32, (1, BE_ROWS), 1) * BT).astype(
        jnp.float32)
    be = jnp.sum((jb >= ends).astype(jnp.int32), axis=0, keepdims=True)
    be = jnp.minimum(be, E - 1)
    nact = (ends[E - 1, 0] * (1.0 / BT)).astype(jnp.int32)
    lanes = lax.broadcasted_iota(jnp.int32, (1, BE_ROWS), 1)
    be_ref[...] = jnp.where(lanes == NB, nact, be)


def _ffn_kernel(be_ref, xs_ref, w1_ref, w2_ref, w3_ref, ys_ref):
    @pl.when(pl.program_id(0) < be_ref[NB])
    def _():
        xb = xs_ref[:, 0:D]
        h1 = jnp.dot(xb, w1_ref[0], preferred_element_type=jnp.float32)
        h2 = jnp.dot(xb, w2_ref[0], preferred_element_type=jnp.float32)
        z = h2 * h1
        hh = 0.5 * z * (1.0 + lax.erf(z * 0.7071067811865476))
        y = jnp.dot(hh, w3_ref[0], preferred_element_type=jnp.float32)
        kwb = xs_ref[:, D : D + 1]
        ys_ref[...] = kwb * y + (1.0 - kwb) * xb


@functools.lru_cache(maxsize=1)
def _make_sc_kernels():
    mesh = plsc.VectorSubcoreMesh(core_axis_name="c", subcore_axis_name="s")

    @functools.partial(
        pl.kernel,
        mesh=mesh,
        out_type=jax.ShapeDtypeStruct((PT, DW), jnp.float32),
        scratch_types=[
            pltpu.VMEM((TW,), jnp.int32),
            pltpu.VMEM((TW, DW), jnp.float32),
            pltpu.SemaphoreType.DMA,
            pltpu.SemaphoreType.DMA,
            pltpu.SemaphoreType.DMA,
        ],
    )
    def sc_scatter(dst_hbm, x_hbm, kw16_hbm, xs_hbm, idx_v, rows_v,
                   sem, sem2, sem3):
        wid = lax.axis_index("s") * NC + lax.axis_index("c")
        base = wid * TW
        c1 = pltpu.async_copy(dst_hbm.at[0, pl.ds(base, TW)], idx_v, sem)
        c2 = pltpu.async_copy(x_hbm.at[pl.ds(base, TW)],
                              rows_v.at[:, pl.ds(0, D)], sem2)
        c3 = pltpu.async_copy(kw16_hbm.at[pl.ds(base, TW)],
                              rows_v.at[:, pl.ds(D, KWW)], sem3)
        c1.wait()
        c2.wait()
        c3.wait()
        pltpu.async_copy(rows_v, xs_hbm.at[idx_v], sem).wait()

    @functools.partial(
        pl.kernel,
        mesh=mesh,
        out_type=jax.ShapeDtypeStruct((T, D), jnp.float32),
        scratch_types=[
            pltpu.VMEM((TW,), jnp.int32),
            pltpu.VMEM((TW, D), jnp.float32),
            pltpu.SemaphoreType.DMA,
        ],
    )
    def sc_gather(dst_hbm, ys_hbm, g_hbm, idx_v, rows_v, sem):
        wid = lax.axis_index("s") * NC + lax.axis_index("c")
        base = wid * TW
        pltpu.sync_copy(dst_hbm.at[0, pl.ds(base, TW)], idx_v)
        pltpu.async_copy(ys_hbm.at[idx_v], rows_v, sem).wait()
        pltpu.sync_copy(rows_v, g_hbm.at[pl.ds(base, TW)])

    return sc_scatter, sc_gather


def kernel(x, Wr, br, w1, w2, w3):
    b, t, d = x.shape
    x_flat = x.reshape(T, D)

    dst2, kw16, be2 = pl.pallas_call(
        _route_kernel,
        out_shape=[
            jax.ShapeDtypeStruct((1, T), jnp.int32),
            jax.ShapeDtypeStruct((T, KWW), jnp.float32),
            jax.ShapeDtypeStruct((1, BE_ROWS), jnp.int32),
        ],
    )(x_flat, Wr, br.reshape(E, 1))

    sc_scatter, sc_gather = _make_sc_kernels()
    xs = sc_scatter(dst2, x_flat, kw16)

    grid_spec = pltpu.PrefetchScalarGridSpec(
        num_scalar_prefetch=1,
        grid=(NB,),
        in_specs=[
            pl.BlockSpec((BT, DW), lambda j, be: (j, 0)),
            pl.BlockSpec((1, D, H), lambda j, be: (be[j], 0, 0)),
            pl.BlockSpec((1, D, H), lambda j, be: (be[j], 0, 0)),
            pl.BlockSpec((1, H, D), lambda j, be: (be[j], 0, 0)),
        ],
        out_specs=pl.BlockSpec((BT, D), lambda j, be: (j, 0)),
    )
    ys = pl.pallas_call(
        _ffn_kernel,
        grid_spec=grid_spec,
        out_shape=jax.ShapeDtypeStruct((PT, D), jnp.float32),
    )(be2.reshape(BE_ROWS), xs, w1, w2, w3)

    out = sc_gather(dst2, ys)
    return out.reshape(b, t, d)


# speedup vs baseline: 1.0467x; 1.0300x over previous
import functools

import jax
import jax.numpy as jnp
from jax import lax
from jax.experimental import pallas as pl
from jax.experimental.pallas import tpu as pltpu
from jax.experimental.pallas import tpu_sc as plsc

E = 8
D = 768
H = 512
T = 2048
CAP = 1024.0
BT = 128
PT = T + E * BT
NB = PT // BT
BE_ROWS = 32
KWW = 128
DW = D + KWW
NC = 2
NS = 16
NW = NC * NS
TW = T // NW


def _cumsum1(a):
    n = a.shape[1]
    d = 1
    while d < n:
        z = jnp.zeros((a.shape[0], d), a.dtype)
        a = a + jnp.concatenate([z, a[:, : n - d]], axis=1)
        d *= 2
    return a


def _route_kernel(x_ref, wr_ref, br_ref, dst_ref, kw_ref, be_ref):
    st = lax.dot_general(
        wr_ref[...], x_ref[...], (((0,), (1,)), ((), ())),
        preferred_element_type=jnp.float32)
    st = st + br_ref[...]
    iota = lax.broadcasted_iota(jnp.int32, (E, T), 0)
    v0 = jnp.max(st, axis=0, keepdims=True)
    e0 = jnp.min(jnp.where(st == v0, iota, E), axis=0, keepdims=True)
    masked = jnp.where(iota == e0, -jnp.inf, st)
    v1 = jnp.max(masked, axis=0, keepdims=True)
    e1 = jnp.min(jnp.where(masked == v1, iota, E), axis=0, keepdims=True)
    s0 = 1.0 / (1.0 + jnp.exp(v1 - v0))
    s1 = 1.0 - s0
    oh0 = (iota == e0).astype(jnp.float32)
    oh1 = (iota == e1).astype(jnp.float32)
    c0 = _cumsum1(oh0)
    c1 = _cumsum1(oh1)
    pos0 = jnp.sum(c0 * oh0, axis=0, keepdims=True)
    pos1 = jnp.sum((c0 + c1) * oh1, axis=0, keepdims=True)
    m0 = pos0 < CAP
    m1 = pos1 < CAP
    kw = jnp.where(m0, s0, 0.0) + jnp.where(m1, s1, 0.0)
    kw_ref[...] = jnp.broadcast_to(jnp.transpose(kw, (1, 0)), (T, KWW))
    g = jnp.maximum(jnp.where(m0, e0, 0), jnp.where(m1, e1, 0))
    ohg = (iota == g).astype(jnp.float32)
    cg = _cumsum1(ohg)
    rank = jnp.sum(cg * ohg, axis=0, keepdims=True) - 1.0
    cnt = cg[:, T - 1 : T]
    pc = jnp.ceil(cnt * (1.0 / BT)) * BT
    tri = (
        lax.broadcasted_iota(jnp.int32, (E, E), 0)
        > lax.broadcasted_iota(jnp.int32, (E, E), 1)
    ).astype(jnp.float32)
    starts = jnp.dot(tri, pc, preferred_element_type=jnp.float32)
    dstf = jnp.sum(ohg * starts, axis=0, keepdims=True) + rank
    dst_ref[...] = dstf.astype(jnp.int32)
    ends = starts + pc
    jb = (lax.broadcasted_iota(jnp.int32, (1, BE_ROWS), 1) * BT).astype(
        jnp.float32)
    be = jnp.sum((jb >= ends).astype(jnp.int32), axis=0, keepdims=True)
    be = jnp.minimum(be, E - 1)
    nact = (ends[E - 1, 0] * (1.0 / BT)).astype(jnp.int32)
    lanes = lax.broadcasted_iota(jnp.int32, (1, BE_ROWS), 1)
    be_ref[...] = jnp.where(lanes == NB, nact, be)


def _ffn_kernel(be_ref, xs_ref, w1_ref, w2_ref, w3_ref, ys_ref):
    @pl.when(pl.program_id(0) < be_ref[NB])
    def _():
        xb = xs_ref[:, 0:D]
        h1 = jnp.dot(xb, w1_ref[0], preferred_element_type=jnp.float32)
        h2 = jnp.dot(xb, w2_ref[0], preferred_element_type=jnp.float32)
        z = h2 * h1
        hh = 0.5 * z * (1.0 + lax.erf(z * 0.7071067811865476))
        y = jnp.dot(hh, w3_ref[0], preferred_element_type=jnp.float32)
        kwb = xs_ref[:, D : D + 1]
        ys_ref[...] = kwb * y + (1.0 - kwb) * xb


@functools.lru_cache(maxsize=1)
def _make_sc_kernels():
    mesh = plsc.VectorSubcoreMesh(core_axis_name="c", subcore_axis_name="s")

    @functools.partial(
        pl.kernel,
        mesh=mesh,
        out_type=jax.ShapeDtypeStruct((PT, DW), jnp.float32),
        scratch_types=[
            pltpu.VMEM((TW,), jnp.int32),
            pltpu.VMEM((TW, DW), jnp.float32),
            pltpu.SemaphoreType.DMA,
            pltpu.SemaphoreType.DMA,
            pltpu.SemaphoreType.DMA,
        ],
    )
    def sc_scatter(dst_hbm, x_hbm, kw16_hbm, xs_hbm, idx_v, rows_v,
                   sem, sem2, sem3):
        wid = lax.axis_index("s") * NC + lax.axis_index("c")
        base = wid * TW
        c1 = pltpu.async_copy(dst_hbm.at[0, pl.ds(base, TW)], idx_v, sem)
        c2 = pltpu.async_copy(x_hbm.at[pl.ds(base, TW)],
                              rows_v.at[:, pl.ds(0, D)], sem2)
        c3 = pltpu.async_copy(kw16_hbm.at[pl.ds(base, TW)],
                              rows_v.at[:, pl.ds(D, KWW)], sem3)
        c1.wait()
        c2.wait()
        c3.wait()
        pltpu.async_copy(rows_v, xs_hbm.at[idx_v], sem).wait()

    @functools.partial(
        pl.kernel,
        mesh=mesh,
        out_type=jax.ShapeDtypeStruct((T, D), jnp.float32),
        scratch_types=[
            pltpu.VMEM((TW,), jnp.int32),
            pltpu.VMEM((TW, D), jnp.float32),
            pltpu.SemaphoreType.DMA,
        ],
    )
    def sc_gather(dst_hbm, ys_hbm, g_hbm, idx_v, rows_v, sem):
        wid = lax.axis_index("s") * NC + lax.axis_index("c")
        base = wid * TW
        pltpu.sync_copy(dst_hbm.at[0, pl.ds(base, TW)], idx_v)
        pltpu.async_copy(ys_hbm.at[idx_v], rows_v, sem).wait()
        pltpu.sync_copy(rows_v, g_hbm.at[pl.ds(base, TW)])

    return sc_scatter, sc_gather


def kernel(x, Wr, br, w1, w2, w3):
    b, t, d = x.shape
    x_flat = x.reshape(T, D)

    dst2, kw16, be2 = pl.pallas_call(
        _route_kernel,
        out_shape=[
            jax.ShapeDtypeStruct((1, T), jnp.int32),
            jax.ShapeDtypeStruct((T, KWW), jnp.float32),
            jax.ShapeDtypeStruct((1, BE_ROWS), jnp.int32),
        ],
    )(x_flat, Wr, br.reshape(E, 1))

    sc_scatter, sc_gather = _make_sc_kernels()
    xs = sc_scatter(dst2, x_flat, kw16)

    def _jm(j, be):
        return jnp.minimum(j, be[NB] - 1)

    grid_spec = pltpu.PrefetchScalarGridSpec(
        num_scalar_prefetch=1,
        grid=(NB,),
        in_specs=[
            pl.BlockSpec((BT, DW), lambda j, be: (_jm(j, be), 0)),
            pl.BlockSpec((1, D, H), lambda j, be: (be[_jm(j, be)], 0, 0)),
            pl.BlockSpec((1, D, H), lambda j, be: (be[_jm(j, be)], 0, 0)),
            pl.BlockSpec((1, H, D), lambda j, be: (be[_jm(j, be)], 0, 0)),
        ],
        out_specs=pl.BlockSpec((BT, D), lambda j, be: (_jm(j, be), 0)),
    )
    ys = pl.pallas_call(
        _ffn_kernel,
        grid_spec=grid_spec,
        out_shape=jax.ShapeDtypeStruct((PT, D), jnp.float32),
    )(be2.reshape(BE_ROWS), xs, w1, w2, w3)

    out = sc_gather(dst2, ys)
    return out.reshape(b, t, d)
